# baseline (speedup 1.0000x reference)
import jax
import jax.numpy as jnp
from jax.experimental import pallas as pl
from jax.experimental.pallas import tpu as pltpu

_B, _N, _C = 8, 8732, 21
_TWO_N = 2 * _N
_ROWS, _LANES = 144, 128
_PADN = _ROWS * _LANES
_CHUNK_ROWS = 8
_NCHUNK = _ROWS // _CHUNK_ROWS
_TOP_K = 200
_CAP = 256
_CONF_T = 0.01
_NMS_T = 0.45


def _prep_kernel(conf_ref, conf2_ref, loc_ref, loc2_ref, dbox_ref,
                 p1_ref, p2_ref, b1_ref, b2_ref):
    def softmax_cmaj(c):
        m = jnp.max(c, axis=0, keepdims=True)
        e = jnp.exp(c - m)
        return e / jnp.sum(e, axis=0, keepdims=True)

    p1_ref[0] = softmax_cmaj(conf_ref[0])[1:, :]
    p2_ref[0] = softmax_cmaj(conf2_ref[0])[1:, :]

    d0 = dbox_ref[0:1, :]
    d1 = dbox_ref[1:2, :]
    d2 = dbox_ref[2:3, :]
    d3 = dbox_ref[3:4, :]

    def decode(loc_r):
        l0 = loc_r[0, 0:1, :]
        l1 = loc_r[0, 1:2, :]
        l2 = loc_r[0, 2:3, :]
        l3 = loc_r[0, 3:4, :]
        cx = d0 + l0 * 0.1 * d2
        cy = d1 + l1 * 0.1 * d3
        w = d2 * jnp.exp(l2 * 0.2)
        h = d3 * jnp.exp(l3 * 0.2)
        return cx - w / 2.0, cy - h / 2.0, cx + w / 2.0, cy + h / 2.0

    x1, y1, x2, y2 = decode(loc_ref)
    b1_ref[0, 0:1, :] = x1
    b1_ref[0, 1:2, :] = y1
    b1_ref[0, 2:3, :] = x2
    b1_ref[0, 3:4, :] = y2

    fx1, fy1, fx2, fy2 = decode(loc2_ref)
    n0 = 1.0 - fx2
    n2 = 1.0 - n0
    b2_ref[0, 0:1, :] = n0
    b2_ref[0, 1:2, :] = fy1
    b2_ref[0, 2:3, :] = n2
    b2_ref[0, 3:4, :] = fy2


def _nms_kernel(scores_ref, boxes_ref, out_ref, keys_ref):
    s = scores_ref[0, 0]
    keys_ref[...] = jnp.where(s > _CONF_T, s, -1.0)

    ridx = jax.lax.broadcasted_iota(jnp.int32, (_ROWS, _LANES), 0)
    lidx = jax.lax.broadcasted_iota(jnp.int32, (_ROWS, _LANES), 1)
    fidx = ridx * _LANES + lidx
    iota = jax.lax.broadcasted_iota(jnp.int32, (1, _CAP), 1)
    li128 = jax.lax.broadcasted_iota(jnp.int32, (1, _LANES), 1)
    rl = (jax.lax.broadcasted_iota(jnp.int32, (_CHUNK_ROWS, _LANES), 0)
          * _LANES
          + jax.lax.broadcasted_iota(jnp.int32, (_CHUNK_ROWS, _LANES), 1))

    k0 = keys_ref[...]
    any1 = jnp.max(jnp.where((k0 >= 0.0) & (fidx < _N), 1.0, 0.0),
                   axis=(0, 1), keepdims=True)

    rm = jnp.full((1, _LANES), -3.0, jnp.float32)
    for c in range(_NCHUNK):
        mc = jnp.max(k0[c * _CHUNK_ROWS:(c + 1) * _CHUNK_ROWS, :],
                     axis=(0, 1), keepdims=True)
        rm = jnp.where(li128 == c, mc, rm)

    def ext_body(t, carry):
        rm, cs, c1, c2, c3, c4 = carry
        m = jnp.max(rm, axis=1, keepdims=True)
        cidx = jnp.max(jnp.where(rm == m, li128, -1),
                       axis=1, keepdims=True)
        base = cidx[0, 0] * _CHUNK_ROWS
        chunk = keys_ref[pl.ds(base, _CHUNK_ROWS), :]
        sel2 = jnp.max(jnp.where(chunk == m, rl, -1),
                       axis=(0, 1), keepdims=True)
        onehot = rl == sel2
        cleared = jnp.where(onehot, -2.0, chunk)
        keys_ref[pl.ds(base, _CHUNK_ROWS), :] = cleared
        newmax = jnp.max(cleared, axis=(0, 1), keepdims=True)
        rm = jnp.where(li128 == cidx, newmax, rm)
        ohf = onehot.astype(jnp.float32)
        g1 = jnp.sum(ohf * boxes_ref[0, 0, pl.ds(base, _CHUNK_ROWS), :],
                     axis=(0, 1), keepdims=True)
        g2 = jnp.sum(ohf * boxes_ref[0, 1, pl.ds(base, _CHUNK_ROWS), :],
                     axis=(0, 1), keepdims=True)
        g3 = jnp.sum(ohf * boxes_ref[0, 2, pl.ds(base, _CHUNK_ROWS), :],
                     axis=(0, 1), keepdims=True)
        g4 = jnp.sum(ohf * boxes_ref[0, 3, pl.ds(base, _CHUNK_ROWS), :],
                     axis=(0, 1), keepdims=True)
        w = iota == t
        cs = jnp.where(w, m, cs)
        c1 = jnp.where(w, g1, c1)
        c2 = jnp.where(w, g2, c2)
        c3 = jnp.where(w, g3, c3)
        c4 = jnp.where(w, g4, c4)
        return rm, cs, c1, c2, c3, c4

    z = jnp.zeros((1, _CAP), jnp.float32)
    _, cs, c1, c2, c3, c4 = jax.lax.fori_loop(
        0, _TOP_K, ext_body, (rm, z - 2.0, z, z, z, z))

    cvalid = (cs >= 0.0).astype(jnp.float32)
    area = (c3 - c1) * (c4 - c2)

    def nms_body(t, state):
        act, o0, o1, o2, o3, o4 = state
        selv = jnp.min(jnp.where(act > 0.5, iota, _CAP),
                       axis=1, keepdims=True)
        oh = iota == selv
        ohf = oh.astype(jnp.float32)
        ss = jnp.sum(ohf * cs, axis=1, keepdims=True)
        sx1 = jnp.sum(ohf * c1, axis=1, keepdims=True)
        sy1 = jnp.sum(ohf * c2, axis=1, keepdims=True)
        sx2 = jnp.sum(ohf * c3, axis=1, keepdims=True)
        sy2 = jnp.sum(ohf * c4, axis=1, keepdims=True)
        sar = jnp.sum(ohf * area, axis=1, keepdims=True)
        hasf = (selv < _CAP).astype(jnp.float32)
        w = iota == t
        o0 = jnp.where(w, ss * hasf, o0)
        o1 = jnp.where(w, sx1 * hasf, o1)
        o2 = jnp.where(w, sy1 * hasf, o2)
        o3 = jnp.where(w, sx2 * hasf, o3)
        o4 = jnp.where(w, sy2 * hasf, o4)
        xx1 = jnp.maximum(c1, sx1)
        yy1 = jnp.maximum(c2, sy1)
        xx2 = jnp.minimum(c3, sx2)
        yy2 = jnp.minimum(c4, sy2)
        iw = jnp.maximum(xx2 - xx1, 0.0)
        ih = jnp.maximum(yy2 - yy1, 0.0)
        inter = iw * ih
        union = area + sar - inter
        iou = inter / jnp.maximum(union, 1e-12)
        suppf = (iou > _NMS_T).astype(jnp.float32)
        act = act * (1.0 - ohf) * (1.0 - hasf * suppf)
        return act, o0, o1, o2, o3, o4

    _, o0, o1, o2, o3, o4 = jax.lax.fori_loop(
        0, _TOP_K, nms_body, (cvalid, z, z, z, z, z))

    out_ref[0, 0, 0:1, :] = o0 * any1
    out_ref[0, 0, 1:2, :] = o1 * any1
    out_ref[0, 0, 2:3, :] = o2 * any1
    out_ref[0, 0, 3:4, :] = o3 * any1
    out_ref[0, 0, 4:5, :] = o4 * any1


def kernel(loc_data, conf_data, loc_data2, conf_data2, dbox_list):
    conf_t = jnp.transpose(conf_data, (0, 2, 1))
    conf2_t = jnp.transpose(conf_data2, (0, 2, 1))
    loc_t = jnp.transpose(loc_data, (0, 2, 1))
    loc2_t = jnp.transpose(loc_data2, (0, 2, 1))
    dbox_t = jnp.transpose(dbox_list, (1, 0))

    p1, p2, b1, b2 = pl.pallas_call(
        _prep_kernel,
        grid=(_B,),
        in_specs=[
            pl.BlockSpec((1, _C, _N), lambda b: (b, 0, 0)),
            pl.BlockSpec((1, _C, _N), lambda b: (b, 0, 0)),
            pl.BlockSpec((1, 4, _N), lambda b: (b, 0, 0)),
            pl.BlockSpec((1, 4, _N), lambda b: (b, 0, 0)),
            pl.BlockSpec((4, _N), lambda b: (0, 0)),
        ],
        out_specs=[
            pl.BlockSpec((1, _C - 1, _N), lambda b: (b, 0, 0)),
            pl.BlockSpec((1, _C - 1, _N), lambda b: (b, 0, 0)),
            pl.BlockSpec((1, 4, _N), lambda b: (b, 0, 0)),
            pl.BlockSpec((1, 4, _N), lambda b: (b, 0, 0)),
        ],
        out_shape=[
            jax.ShapeDtypeStruct((_B, _C - 1, _N), jnp.float32),
            jax.ShapeDtypeStruct((_B, _C - 1, _N), jnp.float32),
            jax.ShapeDtypeStruct((_B, 4, _N), jnp.float32),
            jax.ShapeDtypeStruct((_B, 4, _N), jnp.float32),
        ],
    )(conf_t, conf2_t, loc_t, loc2_t, dbox_t)

    scores = jnp.concatenate([p1, p2], axis=-1)
    scores = jnp.pad(scores, ((0, 0), (0, 0), (0, _PADN - _TWO_N)))
    scores = scores.reshape(_B, _C - 1, _ROWS, _LANES)
    boxes = jnp.concatenate([b1, b2], axis=-1)
    boxes = jnp.pad(boxes, ((0, 0), (0, 0), (0, _PADN - _TWO_N)))
    boxes = boxes.reshape(_B, 4, _ROWS, _LANES)

    outp = pl.pallas_call(
        _nms_kernel,
        grid=(_B, _C - 1),
        in_specs=[
            pl.BlockSpec((1, 1, _ROWS, _LANES), lambda b, c: (b, c, 0, 0)),
            pl.BlockSpec((1, 4, _ROWS, _LANES), lambda b, c: (b, 0, 0, 0)),
        ],
        out_specs=pl.BlockSpec((1, 1, 5, _CAP), lambda b, c: (b, c, 0, 0)),
        out_shape=jax.ShapeDtypeStruct((_B, _C - 1, 5, _CAP), jnp.float32),
        scratch_shapes=[pltpu.VMEM((_ROWS, _LANES), jnp.float32)],
    )(scores, boxes)

    rows = jnp.transpose(outp, (0, 1, 3, 2))[:, :, :_TOP_K, :]
    zeros_cls0 = jnp.zeros((_B, 1, _TOP_K, 5), jnp.float32)
    return jnp.concatenate([zeros_cls0, rows], axis=1)


# speedup vs baseline: 2.4307x; 2.4307x over previous
import jax
import jax.numpy as jnp
from jax.experimental import pallas as pl
from jax.experimental.pallas import tpu as pltpu

_B, _N, _C = 8, 8732, 21
_TWO_N = 2 * _N
_ROWS, _LANES = 144, 128
_PADN = _ROWS * _LANES
_CHUNK_ROWS = 8
_NCHUNK = _ROWS // _CHUNK_ROWS
_TOP_K = 200
_CAP = 256
_CONF_T = 0.01
_NMS_T = 0.45


def _prep_kernel(conf_ref, conf2_ref, loc_ref, loc2_ref, dbox_ref,
                 p1_ref, p2_ref, b1_ref, b2_ref):
    def softmax_cmaj(c):
        m = jnp.max(c, axis=0, keepdims=True)
        e = jnp.exp(c - m)
        return e / jnp.sum(e, axis=0, keepdims=True)

    p1_ref[0] = softmax_cmaj(conf_ref[0])[1:, :]
    p2_ref[0] = softmax_cmaj(conf2_ref[0])[1:, :]

    d0 = dbox_ref[0:1, :]
    d1 = dbox_ref[1:2, :]
    d2 = dbox_ref[2:3, :]
    d3 = dbox_ref[3:4, :]

    def decode(loc_r):
        l0 = loc_r[0, 0:1, :]
        l1 = loc_r[0, 1:2, :]
        l2 = loc_r[0, 2:3, :]
        l3 = loc_r[0, 3:4, :]
        cx = d0 + l0 * 0.1 * d2
        cy = d1 + l1 * 0.1 * d3
        w = d2 * jnp.exp(l2 * 0.2)
        h = d3 * jnp.exp(l3 * 0.2)
        return cx - w / 2.0, cy - h / 2.0, cx + w / 2.0, cy + h / 2.0

    x1, y1, x2, y2 = decode(loc_ref)
    b1_ref[0, 0:1, :] = x1
    b1_ref[0, 1:2, :] = y1
    b1_ref[0, 2:3, :] = x2
    b1_ref[0, 3:4, :] = y2

    fx1, fy1, fx2, fy2 = decode(loc2_ref)
    n0 = 1.0 - fx2
    n2 = 1.0 - n0
    b2_ref[0, 0:1, :] = n0
    b2_ref[0, 1:2, :] = fy1
    b2_ref[0, 2:3, :] = n2
    b2_ref[0, 3:4, :] = fy2


def _nms_kernel(scores_ref, boxes_ref,
                o0_ref, o1_ref, o2_ref, o3_ref, o4_ref, keys_ref):
    P = _C - 1
    li128 = jax.lax.broadcasted_iota(jnp.int32, (1, _LANES), 1)
    riP = jax.lax.broadcasted_iota(jnp.int32, (P, 1), 0)
    iota = jax.lax.broadcasted_iota(jnp.int32, (1, _CAP), 1)
    rl = (jax.lax.broadcasted_iota(jnp.int32, (_CHUNK_ROWS, _LANES), 0)
          * _LANES
          + jax.lax.broadcasted_iota(jnp.int32, (_CHUNK_ROWS, _LANES), 1))
    ridx = jax.lax.broadcasted_iota(jnp.int32, (_ROWS, _LANES), 0)
    lidx = jax.lax.broadcasted_iota(jnp.int32, (_ROWS, _LANES), 1)
    firstset = (ridx * _LANES + lidx) < _N

    rm = jnp.full((P, _LANES), -3.0, jnp.float32)
    a1 = jnp.zeros((P, 1), jnp.float32)
    for p in range(P):
        s = scores_ref[0, p]
        k0 = jnp.where(s > _CONF_T, s, -1.0)
        keys_ref[p] = k0
        a1p = jnp.max(jnp.where((k0 >= 0.0) & firstset, 1.0, 0.0),
                      axis=(0, 1), keepdims=True)
        a1 = jnp.where(riP == p, a1p, a1)
        for c in range(_NCHUNK):
            mc = jnp.max(k0[c * _CHUNK_ROWS:(c + 1) * _CHUNK_ROWS, :],
                         axis=(0, 1), keepdims=True)
            rm = jnp.where((riP == p) & (li128 == c), mc, rm)

    def ext_body(t, carry):
        rm, cs, c1, c2, c3, c4 = carry
        m = jnp.max(rm, axis=1, keepdims=True)
        cidx = jnp.max(jnp.where(rm == m, li128, -1),
                       axis=1, keepdims=True)
        nm_col = jnp.zeros((P, 1), jnp.float32)
        g1c = jnp.zeros((P, 1), jnp.float32)
        g2c = jnp.zeros((P, 1), jnp.float32)
        g3c = jnp.zeros((P, 1), jnp.float32)
        g4c = jnp.zeros((P, 1), jnp.float32)
        for p in range(P):
            m_p = m[p:p + 1, :]
            base = cidx[p, 0] * _CHUNK_ROWS
            chunk = keys_ref[p, pl.ds(base, _CHUNK_ROWS), :]
            sel2 = jnp.max(jnp.where(chunk == m_p, rl, -1),
                           axis=(0, 1), keepdims=True)
            oh = rl == sel2
            cleared = jnp.where(oh, -2.0, chunk)
            keys_ref[p, pl.ds(base, _CHUNK_ROWS), :] = cleared
            nmx = jnp.max(cleared, axis=(0, 1), keepdims=True)
            ohf = oh.astype(jnp.float32)
            w_p = riP == p
            nm_col = jnp.where(w_p, nmx, nm_col)
            g1c = jnp.where(w_p, jnp.sum(
                ohf * boxes_ref[0, 0, pl.ds(base, _CHUNK_ROWS), :],
                axis=(0, 1), keepdims=True), g1c)
            g2c = jnp.where(w_p, jnp.sum(
                ohf * boxes_ref[0, 1, pl.ds(base, _CHUNK_ROWS), :],
                axis=(0, 1), keepdims=True), g2c)
            g3c = jnp.where(w_p, jnp.sum(
                ohf * boxes_ref[0, 2, pl.ds(base, _CHUNK_ROWS), :],
                axis=(0, 1), keepdims=True), g3c)
            g4c = jnp.where(w_p, jnp.sum(
                ohf * boxes_ref[0, 3, pl.ds(base, _CHUNK_ROWS), :],
                axis=(0, 1), keepdims=True), g4c)
        rm = jnp.where(li128 == cidx, nm_col, rm)
        w = iota == t
        cs = jnp.where(w, m, cs)
        c1 = jnp.where(w, g1c, c1)
        c2 = jnp.where(w, g2c, c2)
        c3 = jnp.where(w, g3c, c3)
        c4 = jnp.where(w, g4c, c4)
        return rm, cs, c1, c2, c3, c4

    z = jnp.zeros((P, _CAP), jnp.float32)
    _, cs, c1, c2, c3, c4 = jax.lax.fori_loop(
        0, _TOP_K, ext_body, (rm, z - 2.0, z, z, z, z))

    cvalid = (cs >= 0.0).astype(jnp.float32)
    area = (c3 - c1) * (c4 - c2)

    def nms_body(t, state):
        act, o0, o1, o2, o3, o4 = state
        selv = jnp.min(jnp.where(act > 0.5, iota, _CAP),
                       axis=1, keepdims=True)
        oh = iota == selv
        ohf = oh.astype(jnp.float32)
        ss = jnp.sum(ohf * cs, axis=1, keepdims=True)
        sx1 = jnp.sum(ohf * c1, axis=1, keepdims=True)
        sy1 = jnp.sum(ohf * c2, axis=1, keepdims=True)
        sx2 = jnp.sum(ohf * c3, axis=1, keepdims=True)
        sy2 = jnp.sum(ohf * c4, axis=1, keepdims=True)
        sar = jnp.sum(ohf * area, axis=1, keepdims=True)
        hasf = (selv < _CAP).astype(jnp.float32)
        w = iota == t
        o0 = jnp.where(w, ss * hasf, o0)
        o1 = jnp.where(w, sx1 * hasf, o1)
        o2 = jnp.where(w, sy1 * hasf, o2)
        o3 = jnp.where(w, sx2 * hasf, o3)
        o4 = jnp.where(w, sy2 * hasf, o4)
        xx1 = jnp.maximum(c1, sx1)
        yy1 = jnp.maximum(c2, sy1)
        xx2 = jnp.minimum(c3, sx2)
        yy2 = jnp.minimum(c4, sy2)
        iw = jnp.maximum(xx2 - xx1, 0.0)
        ih = jnp.maximum(yy2 - yy1, 0.0)
        inter = iw * ih
        union = area + sar - inter
        iou = inter / jnp.maximum(union, 1e-12)
        suppf = (iou > _NMS_T).astype(jnp.float32)
        act = act * (1.0 - ohf) * (1.0 - hasf * suppf)
        return act, o0, o1, o2, o3, o4

    _, o0, o1, o2, o3, o4 = jax.lax.fori_loop(
        0, _TOP_K, nms_body, (cvalid, z, z, z, z, z))

    o0_ref[0] = o0 * a1
    o1_ref[0] = o1 * a1
    o2_ref[0] = o2 * a1
    o3_ref[0] = o3 * a1
    o4_ref[0] = o4 * a1


def kernel(loc_data, conf_data, loc_data2, conf_data2, dbox_list):
    conf_t = jnp.transpose(conf_data, (0, 2, 1))
    conf2_t = jnp.transpose(conf_data2, (0, 2, 1))
    loc_t = jnp.transpose(loc_data, (0, 2, 1))
    loc2_t = jnp.transpose(loc_data2, (0, 2, 1))
    dbox_t = jnp.transpose(dbox_list, (1, 0))

    p1, p2, b1, b2 = pl.pallas_call(
        _prep_kernel,
        grid=(_B,),
        in_specs=[
            pl.BlockSpec((1, _C, _N), lambda b: (b, 0, 0)),
            pl.BlockSpec((1, _C, _N), lambda b: (b, 0, 0)),
            pl.BlockSpec((1, 4, _N), lambda b: (b, 0, 0)),
            pl.BlockSpec((1, 4, _N), lambda b: (b, 0, 0)),
            pl.BlockSpec((4, _N), lambda b: (0, 0)),
        ],
        out_specs=[
            pl.BlockSpec((1, _C - 1, _N), lambda b: (b, 0, 0)),
            pl.BlockSpec((1, _C - 1, _N), lambda b: (b, 0, 0)),
            pl.BlockSpec((1, 4, _N), lambda b: (b, 0, 0)),
            pl.BlockSpec((1, 4, _N), lambda b: (b, 0, 0)),
        ],
        out_shape=[
            jax.ShapeDtypeStruct((_B, _C - 1, _N), jnp.float32),
            jax.ShapeDtypeStruct((_B, _C - 1, _N), jnp.float32),
            jax.ShapeDtypeStruct((_B, 4, _N), jnp.float32),
            jax.ShapeDtypeStruct((_B, 4, _N), jnp.float32),
        ],
    )(conf_t, conf2_t, loc_t, loc2_t, dbox_t)

    scores = jnp.concatenate([p1, p2], axis=-1)
    scores = jnp.pad(scores, ((0, 0), (0, 0), (0, _PADN - _TWO_N)))
    scores = scores.reshape(_B, _C - 1, _ROWS, _LANES)
    boxes = jnp.concatenate([b1, b2], axis=-1)
    boxes = jnp.pad(boxes, ((0, 0), (0, 0), (0, _PADN - _TWO_N)))
    boxes = boxes.reshape(_B, 4, _ROWS, _LANES)

    o_spec = pl.BlockSpec((1, _C - 1, _CAP), lambda b: (b, 0, 0))
    o_shape = jax.ShapeDtypeStruct((_B, _C - 1, _CAP), jnp.float32)
    o0, o1, o2, o3, o4 = pl.pallas_call(
        _nms_kernel,
        grid=(_B,),
        in_specs=[
            pl.BlockSpec((1, _C - 1, _ROWS, _LANES), lambda b: (b, 0, 0, 0)),
            pl.BlockSpec((1, 4, _ROWS, _LANES), lambda b: (b, 0, 0, 0)),
        ],
        out_specs=[o_spec, o_spec, o_spec, o_spec, o_spec],
        out_shape=[o_shape, o_shape, o_shape, o_shape, o_shape],
        scratch_shapes=[
            pltpu.VMEM((_C - 1, _ROWS, _LANES), jnp.float32)],
    )(scores, boxes)

    rows = jnp.stack([o0, o1, o2, o3, o4], axis=-1)[:, :, :_TOP_K, :]
    zeros_cls0 = jnp.zeros((_B, 1, _TOP_K, 5), jnp.float32)
    return jnp.concatenate([zeros_cls0, rows], axis=1)
